# Initial kernel scaffold; baseline (speedup 1.0000x reference)
import functools

import jax
import jax.numpy as jnp
from jax import lax
from jax.experimental import pallas as pl
from jax.experimental.pallas import tpu as pltpu
from jax.experimental.pallas import tpu_sc as plsc

N = 10000
E = 320000
NC, NS = 2, 16
NW = NC * NS
B = 128
K = -(-E // (NW * B))
EPAD = NW * K * B
NROWS = 10240
RPT = NROWS // NS
RC = RPT // B

_MESH = plsc.VectorSubcoreMesh(core_axis_name="c", subcore_axis_name="s")


def _make_agg(F):
    @functools.partial(
        pl.kernel,
        out_type=jax.ShapeDtypeStruct((NC, NROWS, F), jnp.float32),
        mesh=_MESH,
        scratch_types=[
            pltpu.VMEM((K, B), jnp.int32),
            pltpu.VMEM((K, B), jnp.int32),
            pltpu.VMEM((2, B, F), jnp.float32),
            pltpu.VMEM((B, F), jnp.float32),
            pltpu.VMEM_SHARED((NROWS, F), jnp.float32),
            pltpu.SemaphoreType.DMA,
        ],
    )
    def agg(rows_hbm, cols_hbm, y_hbm, zeros_hbm, out_hbm,
            idxr, idxc, gbuf, zbuf, acc, sem):
        c = lax.axis_index("c")
        s = lax.axis_index("s")
        w = c * NS + s
        pltpu.sync_copy(rows_hbm.at[w], idxr)
        pltpu.sync_copy(cols_hbm.at[w], idxc)
        pltpu.sync_copy(zeros_hbm, zbuf)
        for i in range(RC):
            pltpu.sync_copy(zbuf, acc.at[pl.ds(s * RPT + i * B, B)])
        plsc.subcore_barrier()

        def body(j, carry):
            pltpu.async_copy(y_hbm.at[idxr.at[j]], gbuf.at[0], sem).wait()
            pltpu.sync_copy(gbuf.at[0], acc.at[idxc.at[j]], add=True)
            return carry

        lax.fori_loop(0, K, body, 0)
        plsc.subcore_barrier()
        for i in range(RC):
            pltpu.sync_copy(acc.at[pl.ds(s * RPT + i * B, B)], gbuf.at[0])
            pltpu.sync_copy(gbuf.at[0],
                            out_hbm.at[c, pl.ds(s * RPT + i * B, B)])

    return agg


_agg64 = _make_agg(64)
_agg32 = _make_agg(32)


@functools.partial(
    pl.kernel,
    out_type=jax.ShapeDtypeStruct((NC, NROWS, 8), jnp.float32),
    mesh=_MESH,
    scratch_types=[
        pltpu.VMEM((K, B), jnp.int32),
        pltpu.VMEM((B, 8), jnp.float32),
        pltpu.VMEM((B, 8), jnp.float32),
        pltpu.VMEM_SHARED((NROWS, 8), jnp.float32),
    ],
)
def _deg(cols_hbm, ones_hbm, zeros_hbm, out_hbm, idxc, ones_v, zbuf, acc):
    c = lax.axis_index("c")
    s = lax.axis_index("s")
    w = c * NS + s
    pltpu.sync_copy(cols_hbm.at[w], idxc)
    pltpu.sync_copy(ones_hbm, ones_v)
    pltpu.sync_copy(zeros_hbm, zbuf)
    for i in range(RC):
        pltpu.sync_copy(zbuf, acc.at[pl.ds(s * RPT + i * B, B)])
    plsc.subcore_barrier()

    def body(j, carry):
        pltpu.sync_copy(ones_v, acc.at[idxc.at[j]], add=True)
        return carry

    lax.fori_loop(0, K, body, 0)
    plsc.subcore_barrier()
    for i in range(RC):
        pltpu.sync_copy(acc.at[pl.ds(s * RPT + i * B, B)], zbuf)
        pltpu.sync_copy(zbuf, out_hbm.at[c, pl.ds(s * RPT + i * B, B)])


def _mm1_body(x_ref, w_ref, d0_ref, d1_ref, y_ref, d_ref):
    d = lax.rsqrt(d0_ref[...] + d1_ref[...] + 1.0)
    xw = jnp.dot(x_ref[...], w_ref[...], preferred_element_type=jnp.float32)
    y_ref[...] = xw * d
    d_ref[...] = d


_mm1 = pl.pallas_call(
    _mm1_body,
    out_shape=(jax.ShapeDtypeStruct((N, 64), jnp.float32),
               jax.ShapeDtypeStruct((N, 1), jnp.float32)),
)


def _mm2_body(p0_ref, p1_ref, y1_ref, d_ref, b_ref, a_ref, w_ref, y2_ref):
    d = d_ref[...]
    t = d * (p0_ref[...] + p1_ref[...] + y1_ref[...]) + b_ref[...]
    h = jnp.where(t >= 0, t, a_ref[0, 0] * t)
    y2_ref[...] = d * jnp.dot(h, w_ref[...],
                              preferred_element_type=jnp.float32)


_mm2 = pl.pallas_call(
    _mm2_body,
    out_shape=jax.ShapeDtypeStruct((N, 32), jnp.float32),
)


def _fin_body(p0_ref, p1_ref, y2_ref, d_ref, b_ref, a_ref, o_ref):
    t = d_ref[...] * (p0_ref[...] + p1_ref[...] + y2_ref[...]) + b_ref[...]
    o_ref[...] = jnp.where(t >= 0, t, a_ref[0, 0] * t)


_fin = pl.pallas_call(
    _fin_body,
    out_shape=jax.ShapeDtypeStruct((N, 32), jnp.float32),
)


def kernel(x, edge_idx, W1, b1, W2, b2, a1, a2):
    row = edge_idx[0].astype(jnp.int32)
    col = edge_idx[1].astype(jnp.int32)
    pad = EPAD - E
    rowp = jnp.concatenate([row, jnp.zeros((pad,), jnp.int32)]).reshape(NW, K, B)
    colp = jnp.concatenate([col, jnp.full((pad,), N, jnp.int32)]).reshape(NW, K, B)
    ones8 = jnp.ones((B, 8), jnp.float32)
    z8 = jnp.zeros((B, 8), jnp.float32)
    z64 = jnp.zeros((B, 64), jnp.float32)
    z32 = jnp.zeros((B, 32), jnp.float32)

    degp = _deg(colp, ones8, z8)
    y1, d = _mm1(x, W1, degp[0, :N, 0:1], degp[1, :N, 0:1])
    p1 = _agg64(rowp, colp, y1, z64)
    y2 = _mm2(p1[0, :N], p1[1, :N], y1, d,
              b1.reshape(1, 64), a1.reshape(1, 1), W2)
    p2 = _agg32(rowp, colp, y2, z32)
    return _fin(p2[0, :N], p2[1, :N], y2, d,
                b2.reshape(1, 32), a2.reshape(1, 1))


# speedup vs baseline: 21.3401x; 21.3401x over previous
import functools

import jax
import jax.numpy as jnp
from jax import lax
from jax.experimental import pallas as pl
from jax.experimental.pallas import tpu as pltpu
from jax.experimental.pallas import tpu_sc as plsc

N = 10000
E = 320000
NC, NS = 2, 16
NW = NC * NS
B = 128
K = -(-E // (NW * B))
EPAD = NW * K * B
NROWS = 10240
RPT = NROWS // NS
RC = RPT // B

_MESH = plsc.VectorSubcoreMesh(core_axis_name="c", subcore_axis_name="s")


def _make_agg(F):
    @functools.partial(
        pl.kernel,
        out_type=jax.ShapeDtypeStruct((NC, NROWS, F), jnp.float32),
        mesh=_MESH,
        compiler_params=pltpu.CompilerParams(use_tc_tiling_on_sc=False),
        scratch_types=[
            pltpu.VMEM((K, B), jnp.int32),
            pltpu.VMEM((K, B), jnp.int32),
            pltpu.VMEM((2, B, F), jnp.float32),
            pltpu.VMEM((B, F), jnp.float32),
            pltpu.VMEM_SHARED((NROWS, F), jnp.float32),
            pltpu.SemaphoreType.DMA,
        ],
    )
    def agg(rows_hbm, cols_hbm, y_hbm, zeros_hbm, out_hbm,
            idxr, idxc, gbuf, zbuf, acc, sem):
        c = lax.axis_index("c")
        s = lax.axis_index("s")
        w = c * NS + s
        pltpu.sync_copy(rows_hbm.at[w], idxr)
        pltpu.sync_copy(cols_hbm.at[w], idxc)
        pltpu.sync_copy(zeros_hbm, zbuf)
        for i in range(RC):
            pltpu.sync_copy(zbuf, acc.at[pl.ds(s * RPT + i * B, B)])
        plsc.subcore_barrier()

        def body(j, carry):
            pltpu.async_copy(y_hbm.at[idxr.at[j]], gbuf.at[0], sem).wait()
            pltpu.sync_copy(gbuf.at[0], acc.at[idxc.at[j]], add=True)
            return carry

        lax.fori_loop(0, K, body, 0)
        plsc.subcore_barrier()
        for i in range(RC):
            pltpu.sync_copy(acc.at[pl.ds(s * RPT + i * B, B)], gbuf.at[0])
            pltpu.sync_copy(gbuf.at[0],
                            out_hbm.at[c, pl.ds(s * RPT + i * B, B)])

    return agg


_agg64 = _make_agg(64)
_agg32 = _make_agg(32)


@functools.partial(
    pl.kernel,
    out_type=jax.ShapeDtypeStruct((NC, NROWS, 8), jnp.float32),
    mesh=_MESH,
    compiler_params=pltpu.CompilerParams(use_tc_tiling_on_sc=False),
    scratch_types=[
        pltpu.VMEM((K, B), jnp.int32),
        pltpu.VMEM((B, 8), jnp.float32),
        pltpu.VMEM((B, 8), jnp.float32),
        pltpu.VMEM_SHARED((NROWS, 8), jnp.float32),
    ],
)
def _deg(cols_hbm, ones_hbm, zeros_hbm, out_hbm, idxc, ones_v, zbuf, acc):
    c = lax.axis_index("c")
    s = lax.axis_index("s")
    w = c * NS + s
    pltpu.sync_copy(cols_hbm.at[w], idxc)
    pltpu.sync_copy(ones_hbm, ones_v)
    pltpu.sync_copy(zeros_hbm, zbuf)
    for i in range(RC):
        pltpu.sync_copy(zbuf, acc.at[pl.ds(s * RPT + i * B, B)])
    plsc.subcore_barrier()

    def body(j, carry):
        pltpu.sync_copy(ones_v, acc.at[idxc.at[j]], add=True)
        return carry

    lax.fori_loop(0, K, body, 0)
    plsc.subcore_barrier()
    for i in range(RC):
        pltpu.sync_copy(acc.at[pl.ds(s * RPT + i * B, B)], zbuf)
        pltpu.sync_copy(zbuf, out_hbm.at[c, pl.ds(s * RPT + i * B, B)])


def _mm1_body(x_ref, w_ref, d0_ref, d1_ref, y_ref, d_ref):
    d = lax.rsqrt(d0_ref[...] + d1_ref[...] + 1.0)
    xw = jnp.dot(x_ref[...], w_ref[...], preferred_element_type=jnp.float32)
    y_ref[...] = xw * d
    d_ref[...] = d


_mm1 = pl.pallas_call(
    _mm1_body,
    out_shape=(jax.ShapeDtypeStruct((N, 64), jnp.float32),
               jax.ShapeDtypeStruct((N, 1), jnp.float32)),
)


def _mm2_body(p0_ref, p1_ref, y1_ref, d_ref, b_ref, a_ref, w_ref, y2_ref):
    d = d_ref[...]
    t = d * (p0_ref[...] + p1_ref[...] + y1_ref[...]) + b_ref[...]
    h = jnp.where(t >= 0, t, a_ref[0, 0] * t)
    y2_ref[...] = d * jnp.dot(h, w_ref[...],
                              preferred_element_type=jnp.float32)


_mm2 = pl.pallas_call(
    _mm2_body,
    out_shape=jax.ShapeDtypeStruct((N, 32), jnp.float32),
)


def _fin_body(p0_ref, p1_ref, y2_ref, d_ref, b_ref, a_ref, o_ref):
    t = d_ref[...] * (p0_ref[...] + p1_ref[...] + y2_ref[...]) + b_ref[...]
    o_ref[...] = jnp.where(t >= 0, t, a_ref[0, 0] * t)


_fin = pl.pallas_call(
    _fin_body,
    out_shape=jax.ShapeDtypeStruct((N, 32), jnp.float32),
)


def kernel(x, edge_idx, W1, b1, W2, b2, a1, a2):
    row = edge_idx[0].astype(jnp.int32)
    col = edge_idx[1].astype(jnp.int32)
    pad = EPAD - E
    rowp = jnp.concatenate([row, jnp.zeros((pad,), jnp.int32)]).reshape(NW, K, B)
    colp = jnp.concatenate([col, jnp.full((pad,), N, jnp.int32)]).reshape(NW, K, B)
    ones8 = jnp.ones((B, 8), jnp.float32)
    z8 = jnp.zeros((B, 8), jnp.float32)
    z64 = jnp.zeros((B, 64), jnp.float32)
    z32 = jnp.zeros((B, 32), jnp.float32)

    degp = _deg(colp, ones8, z8)
    y1, d = _mm1(x, W1, degp[0, :N, 0:1], degp[1, :N, 0:1])
    p1 = _agg64(rowp, colp, y1, z64)
    y2 = _mm2(p1[0, :N], p1[1, :N], y1, d,
              b1.reshape(1, 64), a1.reshape(1, 1), W2)
    p2 = _agg32(rowp, colp, y2, z32)
    return _fin(p2[0, :N], p2[1, :N], y2, d,
                b2.reshape(1, 32), a2.reshape(1, 1))
